# baseline (speedup 1.0000x reference)
import functools

import jax
import jax.numpy as jnp
from jax import lax
from jax.experimental import pallas as pl
from jax.experimental.pallas import tpu as pltpu

NK = 5
BM = 256


def _dot_t(a, b, prec=None):
    return lax.dot_general(
        a, b, (((1,), (1,)), ((), ())),
        preferred_element_type=jnp.float32, precision=prec)


def _body(wf_ref, bf_ref, stm_ref, w0_ref, b0_ref, w1_ref, b1_ref,
          w2_ref, b2_ref, w3_ref, b3_ref, out_ref, acc_w, acc_b,
          *, num_features, bk, nk, bm):
    k = pl.program_id(0)
    i = pl.program_id(1)

    @pl.when(k == 0)
    def _init():
        acc_w[...] = jnp.zeros_like(acc_w)
        acc_b[...] = jnp.zeros_like(acc_b)

    row = pl.ds(i * bm, bm)

    @pl.when(k < nk - 1)
    def _accum_full():
        w0 = w0_ref[...]
        acc_w[row, :] += _dot_t(wf_ref[...], w0)
        acc_b[row, :] += _dot_t(bf_ref[...], w0)

    @pl.when(k == nk - 1)
    def _accum_tail_and_finish():
        valid = num_features - (nk - 1) * bk
        if valid < bk:
            colx = lax.broadcasted_iota(jnp.int32, wf_ref.shape, 1)
            colw = lax.broadcasted_iota(jnp.int32, w0_ref.shape, 1)
            xw = jnp.where(colx < valid, wf_ref[...], 0.0)
            xb = jnp.where(colx < valid, bf_ref[...], 0.0)
            w0 = jnp.where(colw < valid, w0_ref[...], 0.0)
        else:
            xw, xb, w0 = wf_ref[...], bf_ref[...], w0_ref[...]
        w = acc_w[row, :] + _dot_t(xw, w0) + b0_ref[...]
        b = acc_b[row, :] + _dot_t(xb, w0) + b0_ref[...]
        stm = stm_ref[...]
        wb = jnp.concatenate([w, b], axis=1)
        bw = jnp.concatenate([b, w], axis=1)
        accum = stm * wb + (1.0 - stm) * bw
        l1_x = jnp.clip(accum, 0.0, 1.0)
        hi = lax.Precision.HIGHEST
        l2_x = jnp.clip(_dot_t(l1_x, w1_ref[...], hi) + b1_ref[...], 0.0, 1.0)
        l3_x = jnp.clip(_dot_t(l2_x, w2_ref[...], hi) + b2_ref[...], 0.0, 1.0)
        out_ref[...] = (jnp.sum(l3_x * w3_ref[...], axis=1, keepdims=True)
                        + b3_ref[0, 0])


def kernel(white_features, black_features, stm, l0_w, l0_b, l1_w, l1_b,
           l2_w, l2_b, l3_w, l3_b):
    B, F = white_features.shape
    M = l0_w.shape[0]
    bm = min(BM, B)
    bk = ((-(-F // NK) + 127) // 128) * 128
    nk = -(-F // bk)
    nb = B // bm

    body = functools.partial(_body, num_features=F, bk=bk, nk=nk, bm=bm)
    out = pl.pallas_call(
        body,
        grid=(nk, nb),
        in_specs=[
            pl.BlockSpec((bm, bk), lambda k, i: (i, k)),
            pl.BlockSpec((bm, bk), lambda k, i: (i, k)),
            pl.BlockSpec((bm, 2 * M), lambda k, i: (i, 0)),
            pl.BlockSpec((M, bk), lambda k, i: (0, k)),
            pl.BlockSpec((1, M), lambda k, i: (0, 0)),
            pl.BlockSpec(l1_w.shape, lambda k, i: (0, 0)),
            pl.BlockSpec((1, l1_w.shape[0]), lambda k, i: (0, 0)),
            pl.BlockSpec(l2_w.shape, lambda k, i: (0, 0)),
            pl.BlockSpec((1, l2_w.shape[0]), lambda k, i: (0, 0)),
            pl.BlockSpec(l3_w.shape, lambda k, i: (0, 0)),
            pl.BlockSpec(memory_space=pltpu.SMEM),
        ],
        out_specs=pl.BlockSpec((bm, l3_w.shape[0]), lambda k, i: (i, 0)),
        out_shape=jax.ShapeDtypeStruct((B, l3_w.shape[0]), jnp.float32),
        scratch_shapes=[
            pltpu.VMEM((B, M), jnp.float32),
            pltpu.VMEM((B, M), jnp.float32),
        ],
        compiler_params=pltpu.CompilerParams(
            dimension_semantics=("arbitrary", "arbitrary"),
        ),
    )(white_features, black_features, stm, l0_w,
      l0_b.reshape(1, -1), l1_w, l1_b.reshape(1, -1),
      l2_w, l2_b.reshape(1, -1), l3_w, l3_b.reshape(1, -1))
    return out
